# Initial kernel scaffold; baseline (speedup 1.0000x reference)
import functools

import jax
import jax.numpy as jnp
from jax import lax
from jax.experimental import pallas as pl
from jax.experimental.pallas import tpu as pltpu
from jax.experimental.pallas import tpu_sc as plsc

N = 10000
E = 160000
D = 256
HALF = 128
CP = 16
NC = 2
NS = 16
NT = NC * NS
LANES = 16

RPT = N // NS
CHUNK = 125
E_T1 = E // NT
C1 = E_T1 // CHUNK
E_T3 = E // NS
C3 = E_T3 // CHUNK
RB = 1000
NR = N // RB

_mesh = plsc.VectorSubcoreMesh(
    core_axis_name="c", subcore_axis_name="s", num_cores=NC, num_subcores=NS
)


def _deg_body(dst_hbm, out_hbm, dstv, degv):
    c = lax.axis_index("c")
    s = lax.axis_index("s")
    wid = s * NC + c

    zer = jnp.zeros((LANES,), jnp.float32)

    def zero(i, carry):
        degv[pl.ds(i * LANES, LANES)] = zer
        return carry

    lax.fori_loop(0, N // LANES, zero, 0)

    pltpu.sync_copy(dst_hbm.at[pl.ds(wid * E_T1, E_T1)], dstv)

    ones = jnp.ones((LANES,), jnp.float32)

    def acc(i, carry):
        idx = dstv[pl.ds(i * LANES, LANES)]
        plsc.addupdate_scatter(degv, [idx], ones)
        return carry

    lax.fori_loop(0, E_T1 // LANES, acc, 0)
    pltpu.sync_copy(degv, out_hbm.at[wid])


_deg_kernel = functools.partial(
    pl.kernel,
    out_type=jax.ShapeDtypeStruct((NT, N), jnp.float32),
    mesh=_mesh,
    scratch_types=[
        pltpu.VMEM((E_T1,), jnp.int32),
        pltpu.VMEM((N,), jnp.float32),
    ],
)(_deg_body)


def _mm1_body(x_ref, w1_ref, degt_ref, hs_ref):
    deg = jnp.sum(degt_ref[...], axis=1, keepdims=True) + 1.0
    dinv = lax.rsqrt(deg)
    prod = jnp.dot(x_ref[...], w1_ref[...], preferred_element_type=jnp.float32)
    hs_ref[...] = (prod * dinv)[None]


def _mm1(x, w1, degt):
    return pl.pallas_call(
        _mm1_body,
        grid=(NR, NC),
        in_specs=[
            pl.BlockSpec((RB, D), lambda r, h: (r, 0)),
            pl.BlockSpec((D, HALF), lambda r, h: (0, h)),
            pl.BlockSpec((RB, NT), lambda r, h: (r, 0)),
        ],
        out_specs=pl.BlockSpec((1, RB, HALF), lambda r, h: (h, r, 0)),
        out_shape=jax.ShapeDtypeStruct((NC, N, HALF), jnp.float32),
    )(x, w1, degt)


def _mp1_body(hs_hbm, src_hbm, dst_hbm, out_hbm, srcv, dstv, buf, acc, sem):
    c = lax.axis_index("c")
    s = lax.axis_index("s")

    pltpu.sync_copy(src_hbm.at[c, s], srcv)
    pltpu.sync_copy(dst_hbm.at[s], dstv)
    pltpu.sync_copy(
        hs_hbm.at[pl.ds(c * N + s * RPT, RPT)], acc.at[pl.ds(s * RPT, RPT)]
    )
    plsc.subcore_barrier()

    def chunk(j, carry):
        pltpu.async_copy(hs_hbm.at[srcv.at[j]], buf, sem).wait()
        pltpu.sync_copy(buf, acc.at[dstv.at[j]], add=True)
        return carry

    lax.fori_loop(0, C3, chunk, 0)
    plsc.subcore_barrier()
    pltpu.sync_copy(
        acc.at[pl.ds(s * RPT, RPT)], out_hbm.at[pl.ds(c * N + s * RPT, RPT)]
    )


_mp1_kernel = functools.partial(
    pl.kernel,
    out_type=jax.ShapeDtypeStruct((NC * N, HALF), jnp.float32),
    mesh=_mesh,
    scratch_types=[
        pltpu.VMEM((C3, CHUNK), jnp.int32),
        pltpu.VMEM((C3, CHUNK), jnp.int32),
        pltpu.VMEM((CHUNK, HALF), jnp.float32),
        pltpu.VMEM_SHARED((N, HALF), jnp.float32),
        pltpu.SemaphoreType.DMA,
    ],
)(_mp1_body)


def _mm2_body(s1_ref, degt_ref, b1_ref, w2_ref, y_ref):
    deg = jnp.sum(degt_ref[...], axis=1, keepdims=True) + 1.0
    dinv = lax.rsqrt(deg)
    s1 = s1_ref[...]
    b1 = b1_ref[...]
    w2 = w2_ref[...]
    ha = jnp.maximum(s1[0] * dinv + b1[0], 0.0)
    hb = jnp.maximum(s1[1] * dinv + b1[1], 0.0)
    y = jnp.dot(ha, w2[0], preferred_element_type=jnp.float32)
    y = y + jnp.dot(hb, w2[1], preferred_element_type=jnp.float32)
    y_ref[...] = y * dinv


def _mm2(s1, degt, b1r, w2r):
    return pl.pallas_call(
        _mm2_body,
        grid=(NR,),
        in_specs=[
            pl.BlockSpec((NC, RB, HALF), lambda r: (0, r, 0)),
            pl.BlockSpec((RB, NT), lambda r: (r, 0)),
            pl.BlockSpec((NC, 1, HALF), lambda r: (0, 0, 0)),
            pl.BlockSpec((NC, HALF, CP), lambda r: (0, 0, 0)),
        ],
        out_specs=pl.BlockSpec((RB, CP), lambda r: (r, 0)),
        out_shape=jax.ShapeDtypeStruct((N, CP), jnp.float32),
    )(s1, degt, b1r, w2r)


def _mp2_body(y_hbm, src_hbm, dst_hbm, out_hbm, srcv, dstv, buf, acc, sem):
    c = lax.axis_index("c")
    s = lax.axis_index("s")
    wid = s * NC + c

    pltpu.sync_copy(src_hbm.at[wid], srcv)
    pltpu.sync_copy(dst_hbm.at[wid], dstv)

    zer = jnp.zeros((LANES,), jnp.float32)

    def zero(i, carry):
        buf[i] = zer
        return carry

    lax.fori_loop(0, CHUNK, zero, 0)
    for k in range(RPT // CHUNK):
        pltpu.sync_copy(buf, acc.at[pl.ds(s * RPT + k * CHUNK, CHUNK)])
    plsc.subcore_barrier()

    def chunk(j, carry):
        pltpu.async_copy(y_hbm.at[srcv.at[j]], buf, sem).wait()
        pltpu.sync_copy(buf, acc.at[dstv.at[j]], add=True)
        return carry

    lax.fori_loop(0, C1, chunk, 0)
    plsc.subcore_barrier()
    pltpu.sync_copy(
        acc.at[pl.ds(s * RPT, RPT)], out_hbm.at[c, pl.ds(s * RPT, RPT)]
    )


_mp2_kernel = functools.partial(
    pl.kernel,
    out_type=jax.ShapeDtypeStruct((NC, N, CP), jnp.float32),
    mesh=_mesh,
    scratch_types=[
        pltpu.VMEM((C1, CHUNK), jnp.int32),
        pltpu.VMEM((C1, CHUNK), jnp.int32),
        pltpu.VMEM((CHUNK, CP), jnp.float32),
        pltpu.VMEM_SHARED((N, CP), jnp.float32),
        pltpu.SemaphoreType.DMA,
    ],
)(_mp2_body)


def _lsm_body(p_ref, y_ref, degt_ref, b2_ref, out_ref):
    deg = jnp.sum(degt_ref[...], axis=1, keepdims=True) + 1.0
    dinv = lax.rsqrt(deg)
    p = p_ref[...]
    tot = p[0] + p[1] + y_ref[...]
    logits = tot * dinv + b2_ref[...]
    col = lax.broadcasted_iota(jnp.int32, logits.shape, 1)
    valid = col < 7
    masked = jnp.where(valid, logits, -1e30)
    m = jnp.max(masked, axis=1, keepdims=True)
    z = logits - m
    e = jnp.where(valid, jnp.exp(z), 0.0)
    ssum = jnp.sum(e, axis=1, keepdims=True)
    out_ref[...] = z - jnp.log(ssum)


def _lsm(p, y, degt, b2r):
    return pl.pallas_call(
        _lsm_body,
        grid=(NR,),
        in_specs=[
            pl.BlockSpec((NC, RB, CP), lambda r: (0, r, 0)),
            pl.BlockSpec((RB, CP), lambda r: (r, 0)),
            pl.BlockSpec((RB, NT), lambda r: (r, 0)),
            pl.BlockSpec((1, CP), lambda r: (0, 0)),
        ],
        out_specs=pl.BlockSpec((RB, CP), lambda r: (r, 0)),
        out_shape=jax.ShapeDtypeStruct((N, CP), jnp.float32),
    )(p, y, degt, b2r)


def kernel(x, edge_index, W1, b1, W2, b2):
    ei = edge_index.astype(jnp.int32)
    src = ei[0]
    dst = ei[1]

    src2 = jnp.stack([src, src + N]).reshape(NC, NS, C3, CHUNK)
    dst3 = dst.reshape(NS, C3, CHUNK)
    srcw = src.reshape(NT, C1, CHUNK)
    dstw = dst.reshape(NT, C1, CHUNK)

    w2p = jnp.pad(W2, ((0, 0), (0, CP - W2.shape[1]))).reshape(NC, HALF, CP)
    b1r = b1.reshape(NC, 1, HALF)
    b2r = jnp.pad(b2, (0, CP - b2.shape[0])).reshape(1, CP)

    degp = _deg_kernel(dst)
    degt = degp.T

    hs = _mm1(x, W1, degt)
    s1 = _mp1_kernel(hs.reshape(NC * N, HALF), src2, dst3)
    y2s = _mm2(s1.reshape(NC, N, HALF), degt, b1r, w2p)
    p = _mp2_kernel(y2s, srcw, dstw)
    out = _lsm(p, y2s, degt, b2r)
    return out[:, :7]


# speedup vs baseline: 16.6775x; 16.6775x over previous
import functools

import jax
import jax.numpy as jnp
from jax import lax
from jax.experimental import pallas as pl
from jax.experimental.pallas import tpu as pltpu
from jax.experimental.pallas import tpu_sc as plsc

N = 10000
E = 160000
D = 256
HALF = 128
CP = 16
NC = 2
NS = 16
NT = NC * NS
LANES = 16

RPT = N // NS
CHUNK = 125
E_T1 = E // NT
C1 = E_T1 // CHUNK
E_T3 = E // NS
C3 = E_T3 // CHUNK
RB = 1000
NR = N // RB

_mesh = plsc.VectorSubcoreMesh(
    core_axis_name="c", subcore_axis_name="s", num_cores=NC, num_subcores=NS
)


def _deg_body(dst_hbm, out_hbm, dstv, degv):
    c = lax.axis_index("c")
    s = lax.axis_index("s")
    wid = s * NC + c

    zer = jnp.zeros((LANES,), jnp.float32)

    def zero(i, carry):
        degv[pl.ds(i * LANES, LANES)] = zer
        return carry

    lax.fori_loop(0, N // LANES, zero, 0)

    pltpu.sync_copy(dst_hbm.at[pl.ds(wid * E_T1, E_T1)], dstv)

    ones = jnp.ones((LANES,), jnp.float32)

    def acc(i, carry):
        idx = dstv[pl.ds(i * LANES, LANES)]
        plsc.addupdate_scatter(degv, [idx], ones)
        return carry

    lax.fori_loop(0, E_T1 // LANES, acc, 0)
    pltpu.sync_copy(degv, out_hbm.at[wid])


_deg_kernel = functools.partial(
    pl.kernel,
    out_type=jax.ShapeDtypeStruct((NT, N), jnp.float32),
    mesh=_mesh,
    scratch_types=[
        pltpu.VMEM((E_T1,), jnp.int32),
        pltpu.VMEM((N,), jnp.float32),
    ],
    compiler_params=pltpu.CompilerParams(needs_layout_passes=False),
)(_deg_body)


def _mm1_body(x_ref, w1_ref, degt_ref, hs_ref):
    deg = jnp.sum(degt_ref[...], axis=1, keepdims=True) + 1.0
    dinv = lax.rsqrt(deg)
    prod = jnp.dot(x_ref[...], w1_ref[...], preferred_element_type=jnp.float32)
    hs_ref[...] = (prod * dinv)[None]


def _mm1(x, w1, degt):
    return pl.pallas_call(
        _mm1_body,
        grid=(NR, NC),
        in_specs=[
            pl.BlockSpec((RB, D), lambda r, h: (r, 0)),
            pl.BlockSpec((D, HALF), lambda r, h: (0, h)),
            pl.BlockSpec((RB, NT), lambda r, h: (r, 0)),
        ],
        out_specs=pl.BlockSpec((1, RB, HALF), lambda r, h: (h, r, 0)),
        out_shape=jax.ShapeDtypeStruct((NC, N, HALF), jnp.float32),
    )(x, w1, degt)


def _mp1_body(hs_hbm, src_hbm, dst_hbm, out_hbm, srcv, dstv, buf, acc, sem):
    c = lax.axis_index("c")
    s = lax.axis_index("s")

    pltpu.sync_copy(src_hbm.at[c, s], srcv)
    pltpu.sync_copy(dst_hbm.at[s], dstv)
    pltpu.sync_copy(
        hs_hbm.at[pl.ds(c * N + s * RPT, RPT)], acc.at[pl.ds(s * RPT, RPT)]
    )
    plsc.subcore_barrier()

    def chunk(j, carry):
        pltpu.async_copy(hs_hbm.at[srcv.at[j]], buf, sem).wait()
        pltpu.sync_copy(buf, acc.at[dstv.at[j]], add=True)
        return carry

    lax.fori_loop(0, C3, chunk, 0)
    plsc.subcore_barrier()
    pltpu.sync_copy(
        acc.at[pl.ds(s * RPT, RPT)], out_hbm.at[pl.ds(c * N + s * RPT, RPT)]
    )


_mp1_kernel = functools.partial(
    pl.kernel,
    out_type=jax.ShapeDtypeStruct((NC * N, HALF), jnp.float32),
    mesh=_mesh,
    scratch_types=[
        pltpu.VMEM((C3, CHUNK), jnp.int32),
        pltpu.VMEM((C3, CHUNK), jnp.int32),
        pltpu.VMEM((CHUNK, HALF), jnp.float32),
        pltpu.VMEM_SHARED((N, HALF), jnp.float32),
        pltpu.SemaphoreType.DMA,
    ],
    compiler_params=pltpu.CompilerParams(use_tc_tiling_on_sc=False),
)(_mp1_body)


def _mm2_body(s1_ref, degt_ref, b1_ref, w2_ref, y_ref):
    deg = jnp.sum(degt_ref[...], axis=1, keepdims=True) + 1.0
    dinv = lax.rsqrt(deg)
    s1 = s1_ref[...]
    b1 = b1_ref[...]
    w2 = w2_ref[...]
    ha = jnp.maximum(s1[0] * dinv + b1[0], 0.0)
    hb = jnp.maximum(s1[1] * dinv + b1[1], 0.0)
    y = jnp.dot(ha, w2[0], preferred_element_type=jnp.float32)
    y = y + jnp.dot(hb, w2[1], preferred_element_type=jnp.float32)
    y_ref[...] = y * dinv


def _mm2(s1, degt, b1r, w2r):
    return pl.pallas_call(
        _mm2_body,
        grid=(NR,),
        in_specs=[
            pl.BlockSpec((NC, RB, HALF), lambda r: (0, r, 0)),
            pl.BlockSpec((RB, NT), lambda r: (r, 0)),
            pl.BlockSpec((NC, 1, HALF), lambda r: (0, 0, 0)),
            pl.BlockSpec((NC, HALF, CP), lambda r: (0, 0, 0)),
        ],
        out_specs=pl.BlockSpec((RB, CP), lambda r: (r, 0)),
        out_shape=jax.ShapeDtypeStruct((N, CP), jnp.float32),
    )(s1, degt, b1r, w2r)


def _mp2_body(y_hbm, src_hbm, dst_hbm, out_hbm, srcv, dstv, buf, acc, sem):
    c = lax.axis_index("c")
    s = lax.axis_index("s")
    wid = s * NC + c

    pltpu.sync_copy(src_hbm.at[wid], srcv)
    pltpu.sync_copy(dst_hbm.at[wid], dstv)

    zer = jnp.zeros((LANES,), jnp.float32)

    def zero(i, carry):
        buf[i] = zer
        return carry

    lax.fori_loop(0, CHUNK, zero, 0)
    for k in range(RPT // CHUNK):
        pltpu.sync_copy(buf, acc.at[pl.ds(s * RPT + k * CHUNK, CHUNK)])
    plsc.subcore_barrier()

    def chunk(j, carry):
        pltpu.async_copy(y_hbm.at[srcv.at[j]], buf, sem).wait()
        pltpu.sync_copy(buf, acc.at[dstv.at[j]], add=True)
        return carry

    lax.fori_loop(0, C1, chunk, 0)
    plsc.subcore_barrier()
    pltpu.sync_copy(
        acc.at[pl.ds(s * RPT, RPT)], out_hbm.at[c, pl.ds(s * RPT, RPT)]
    )


_mp2_kernel = functools.partial(
    pl.kernel,
    out_type=jax.ShapeDtypeStruct((NC, N, CP), jnp.float32),
    mesh=_mesh,
    scratch_types=[
        pltpu.VMEM((C1, CHUNK), jnp.int32),
        pltpu.VMEM((C1, CHUNK), jnp.int32),
        pltpu.VMEM((CHUNK, CP), jnp.float32),
        pltpu.VMEM_SHARED((N, CP), jnp.float32),
        pltpu.SemaphoreType.DMA,
    ],
    compiler_params=pltpu.CompilerParams(use_tc_tiling_on_sc=False),
)(_mp2_body)


def _lsm_body(p_ref, y_ref, degt_ref, b2_ref, out_ref):
    deg = jnp.sum(degt_ref[...], axis=1, keepdims=True) + 1.0
    dinv = lax.rsqrt(deg)
    p = p_ref[...]
    tot = p[0] + p[1] + y_ref[...]
    logits = tot * dinv + b2_ref[...]
    col = lax.broadcasted_iota(jnp.int32, logits.shape, 1)
    valid = col < 7
    masked = jnp.where(valid, logits, -1e30)
    m = jnp.max(masked, axis=1, keepdims=True)
    z = logits - m
    e = jnp.where(valid, jnp.exp(z), 0.0)
    ssum = jnp.sum(e, axis=1, keepdims=True)
    out_ref[...] = z - jnp.log(ssum)


def _lsm(p, y, degt, b2r):
    return pl.pallas_call(
        _lsm_body,
        grid=(NR,),
        in_specs=[
            pl.BlockSpec((NC, RB, CP), lambda r: (0, r, 0)),
            pl.BlockSpec((RB, CP), lambda r: (r, 0)),
            pl.BlockSpec((RB, NT), lambda r: (r, 0)),
            pl.BlockSpec((1, CP), lambda r: (0, 0)),
        ],
        out_specs=pl.BlockSpec((RB, CP), lambda r: (r, 0)),
        out_shape=jax.ShapeDtypeStruct((N, CP), jnp.float32),
    )(p, y, degt, b2r)


def kernel(x, edge_index, W1, b1, W2, b2):
    ei = edge_index.astype(jnp.int32)
    src = ei[0]
    dst = ei[1]

    src2 = jnp.stack([src, src + N]).reshape(NC, NS, C3, CHUNK)
    dst3 = dst.reshape(NS, C3, CHUNK)
    srcw = src.reshape(NT, C1, CHUNK)
    dstw = dst.reshape(NT, C1, CHUNK)

    w2p = jnp.pad(W2, ((0, 0), (0, CP - W2.shape[1]))).reshape(NC, HALF, CP)
    b1r = b1.reshape(NC, 1, HALF)
    b2r = jnp.pad(b2, (0, CP - b2.shape[0])).reshape(1, CP)

    degp = _deg_kernel(dst)
    degt = degp.T

    hs = _mm1(x, W1, degt)
    s1 = _mp1_kernel(hs.reshape(NC * N, HALF), src2, dst3)
    y2s = _mm2(s1.reshape(NC, N, HALF), degt, b1r, w2p)
    p = _mp2_kernel(y2s, srcw, dstw)
    out = _lsm(p, y2s, degt, b2r)
    return out[:, :7]


# speedup vs baseline: 18.6877x; 1.1205x over previous
import functools

import jax
import jax.numpy as jnp
from jax import lax
from jax.experimental import pallas as pl
from jax.experimental.pallas import tpu as pltpu
from jax.experimental.pallas import tpu_sc as plsc

N = 10000
E = 160000
D = 256
HALF = 128
CP = 16
NC = 2
NS = 16
NT = NC * NS
LANES = 16

RPT = N // NS
CHUNK = 100
E_T1 = E // NT
C1 = E_T1 // CHUNK
E_T3 = E // NS
C3 = E_T3 // CHUNK
RB = 1000
NR = N // RB

_mesh = plsc.VectorSubcoreMesh(
    core_axis_name="c", subcore_axis_name="s", num_cores=NC, num_subcores=NS
)


def _deg_body(dst_hbm, out_hbm, dstv, degv):
    c = lax.axis_index("c")
    s = lax.axis_index("s")
    wid = s * NC + c

    zer = jnp.zeros((LANES,), jnp.float32)

    def zero(i, carry):
        degv[pl.ds(i * LANES, LANES)] = zer
        return carry

    lax.fori_loop(0, N // LANES, zero, 0)

    pltpu.sync_copy(dst_hbm.at[pl.ds(wid * E_T1, E_T1)], dstv)

    ones = jnp.ones((LANES,), jnp.float32)

    def acc(i, carry):
        idx = dstv[pl.ds(i * LANES, LANES)]
        plsc.addupdate_scatter(degv, [idx], ones)
        return carry

    lax.fori_loop(0, E_T1 // LANES, acc, 0)
    pltpu.sync_copy(degv, out_hbm.at[wid])


_deg_kernel = functools.partial(
    pl.kernel,
    out_type=jax.ShapeDtypeStruct((NT, N), jnp.float32),
    mesh=_mesh,
    scratch_types=[
        pltpu.VMEM((E_T1,), jnp.int32),
        pltpu.VMEM((N,), jnp.float32),
    ],
    compiler_params=pltpu.CompilerParams(needs_layout_passes=False),
)(_deg_body)


def _mm1_body(x_ref, w1_ref, degt_ref, hs_ref):
    deg = jnp.sum(degt_ref[...], axis=1, keepdims=True) + 1.0
    dinv = lax.rsqrt(deg)
    prod = jnp.dot(x_ref[...], w1_ref[...], preferred_element_type=jnp.float32)
    hs_ref[...] = (prod * dinv)[None]


def _mm1(x, w1, degt):
    return pl.pallas_call(
        _mm1_body,
        grid=(NR, NC),
        in_specs=[
            pl.BlockSpec((RB, D), lambda r, h: (r, 0)),
            pl.BlockSpec((D, HALF), lambda r, h: (0, h)),
            pl.BlockSpec((RB, NT), lambda r, h: (r, 0)),
        ],
        out_specs=pl.BlockSpec((1, RB, HALF), lambda r, h: (h, r, 0)),
        out_shape=jax.ShapeDtypeStruct((NC, N, HALF), jnp.float32),
    )(x, w1, degt)


def _gs_pipeline(rows_hbm, srcv, dstv, b0, b1, acc, sem0, sem1, nchunks):
    pltpu.async_copy(rows_hbm.at[srcv.at[0]], b0, sem0)

    def step(jj, carry):
        j = 2 * jj
        pltpu.make_async_copy(rows_hbm.at[srcv.at[j]], b0, sem0).wait()
        pltpu.async_copy(rows_hbm.at[srcv.at[j + 1]], b1, sem1)
        pltpu.sync_copy(b0, acc.at[dstv.at[j]], add=True)
        pltpu.make_async_copy(rows_hbm.at[srcv.at[j + 1]], b1, sem1).wait()

        @pl.when(jj < nchunks // 2 - 1)
        def _():
            pltpu.async_copy(rows_hbm.at[srcv.at[j + 2]], b0, sem0)

        pltpu.sync_copy(b1, acc.at[dstv.at[j + 1]], add=True)
        return carry

    lax.fori_loop(0, nchunks // 2, step, 0)


def _mp1_body(hs_hbm, src_hbm, dst_hbm, out_hbm, srcv, dstv, b0, b1, acc,
              sem0, sem1):
    c = lax.axis_index("c")
    s = lax.axis_index("s")

    pltpu.sync_copy(src_hbm.at[c, s], srcv)
    pltpu.sync_copy(dst_hbm.at[s], dstv)
    pltpu.sync_copy(
        hs_hbm.at[pl.ds(c * N + s * RPT, RPT)], acc.at[pl.ds(s * RPT, RPT)]
    )
    plsc.subcore_barrier()
    _gs_pipeline(hs_hbm, srcv, dstv, b0, b1, acc, sem0, sem1, C3)
    plsc.subcore_barrier()
    pltpu.sync_copy(
        acc.at[pl.ds(s * RPT, RPT)], out_hbm.at[pl.ds(c * N + s * RPT, RPT)]
    )


_mp1_kernel = functools.partial(
    pl.kernel,
    out_type=jax.ShapeDtypeStruct((NC * N, HALF), jnp.float32),
    mesh=_mesh,
    scratch_types=[
        pltpu.VMEM((C3, CHUNK), jnp.int32),
        pltpu.VMEM((C3, CHUNK), jnp.int32),
        pltpu.VMEM((CHUNK, HALF), jnp.float32),
        pltpu.VMEM((CHUNK, HALF), jnp.float32),
        pltpu.VMEM_SHARED((N, HALF), jnp.float32),
        pltpu.SemaphoreType.DMA,
        pltpu.SemaphoreType.DMA,
    ],
    compiler_params=pltpu.CompilerParams(use_tc_tiling_on_sc=False),
)(_mp1_body)


def _mm2_body(s1_ref, degt_ref, b1_ref, w2_ref, y_ref):
    deg = jnp.sum(degt_ref[...], axis=1, keepdims=True) + 1.0
    dinv = lax.rsqrt(deg)
    s1 = s1_ref[...]
    b1 = b1_ref[...]
    w2 = w2_ref[...]
    ha = jnp.maximum(s1[0] * dinv + b1[0], 0.0)
    hb = jnp.maximum(s1[1] * dinv + b1[1], 0.0)
    y = jnp.dot(ha, w2[0], preferred_element_type=jnp.float32)
    y = y + jnp.dot(hb, w2[1], preferred_element_type=jnp.float32)
    y_ref[...] = y * dinv


def _mm2(s1, degt, b1r, w2r):
    return pl.pallas_call(
        _mm2_body,
        grid=(NR,),
        in_specs=[
            pl.BlockSpec((NC, RB, HALF), lambda r: (0, r, 0)),
            pl.BlockSpec((RB, NT), lambda r: (r, 0)),
            pl.BlockSpec((NC, 1, HALF), lambda r: (0, 0, 0)),
            pl.BlockSpec((NC, HALF, CP), lambda r: (0, 0, 0)),
        ],
        out_specs=pl.BlockSpec((RB, CP), lambda r: (r, 0)),
        out_shape=jax.ShapeDtypeStruct((N, CP), jnp.float32),
    )(s1, degt, b1r, w2r)


def _mp2_body(y_hbm, src_hbm, dst_hbm, out_hbm, srcv, dstv, b0, b1, acc,
              sem0, sem1):
    c = lax.axis_index("c")
    s = lax.axis_index("s")
    wid = s * NC + c

    pltpu.sync_copy(src_hbm.at[wid], srcv)
    pltpu.sync_copy(dst_hbm.at[wid], dstv)

    zer = jnp.zeros((LANES,), jnp.float32)

    def zero(i, carry):
        b0[i] = zer
        return carry

    lax.fori_loop(0, CHUNK, zero, 0)
    full, rem = divmod(RPT, CHUNK)
    for k in range(full):
        pltpu.sync_copy(b0, acc.at[pl.ds(s * RPT + k * CHUNK, CHUNK)])
    if rem:
        pltpu.sync_copy(
            b0.at[pl.ds(0, rem)],
            acc.at[pl.ds(s * RPT + full * CHUNK, rem)],
        )
    plsc.subcore_barrier()
    _gs_pipeline(y_hbm, srcv, dstv, b0, b1, acc, sem0, sem1, C1)
    plsc.subcore_barrier()
    pltpu.sync_copy(
        acc.at[pl.ds(s * RPT, RPT)], out_hbm.at[c, pl.ds(s * RPT, RPT)]
    )


_mp2_kernel = functools.partial(
    pl.kernel,
    out_type=jax.ShapeDtypeStruct((NC, N, CP), jnp.float32),
    mesh=_mesh,
    scratch_types=[
        pltpu.VMEM((C1, CHUNK), jnp.int32),
        pltpu.VMEM((C1, CHUNK), jnp.int32),
        pltpu.VMEM((CHUNK, CP), jnp.float32),
        pltpu.VMEM((CHUNK, CP), jnp.float32),
        pltpu.VMEM_SHARED((N, CP), jnp.float32),
        pltpu.SemaphoreType.DMA,
        pltpu.SemaphoreType.DMA,
    ],
    compiler_params=pltpu.CompilerParams(use_tc_tiling_on_sc=False),
)(_mp2_body)


def _lsm_body(p_ref, y_ref, degt_ref, b2_ref, out_ref):
    deg = jnp.sum(degt_ref[...], axis=1, keepdims=True) + 1.0
    dinv = lax.rsqrt(deg)
    p = p_ref[...]
    tot = p[0] + p[1] + y_ref[...]
    logits = tot * dinv + b2_ref[...]
    col = lax.broadcasted_iota(jnp.int32, logits.shape, 1)
    valid = col < 7
    masked = jnp.where(valid, logits, -1e30)
    m = jnp.max(masked, axis=1, keepdims=True)
    z = logits - m
    e = jnp.where(valid, jnp.exp(z), 0.0)
    ssum = jnp.sum(e, axis=1, keepdims=True)
    out_ref[...] = z - jnp.log(ssum)


def _lsm(p, y, degt, b2r):
    return pl.pallas_call(
        _lsm_body,
        grid=(NR,),
        in_specs=[
            pl.BlockSpec((NC, RB, CP), lambda r: (0, r, 0)),
            pl.BlockSpec((RB, CP), lambda r: (r, 0)),
            pl.BlockSpec((RB, NT), lambda r: (r, 0)),
            pl.BlockSpec((1, CP), lambda r: (0, 0)),
        ],
        out_specs=pl.BlockSpec((RB, CP), lambda r: (r, 0)),
        out_shape=jax.ShapeDtypeStruct((N, CP), jnp.float32),
    )(p, y, degt, b2r)


def kernel(x, edge_index, W1, b1, W2, b2):
    ei = edge_index.astype(jnp.int32)
    src = ei[0]
    dst = ei[1]

    src2 = jnp.stack([src, src + N]).reshape(NC, NS, C3, CHUNK)
    dst3 = dst.reshape(NS, C3, CHUNK)
    srcw = src.reshape(NT, C1, CHUNK)
    dstw = dst.reshape(NT, C1, CHUNK)

    w2p = jnp.pad(W2, ((0, 0), (0, CP - W2.shape[1]))).reshape(NC, HALF, CP)
    b1r = b1.reshape(NC, 1, HALF)
    b2r = jnp.pad(b2, (0, CP - b2.shape[0])).reshape(1, CP)

    degp = _deg_kernel(dst)
    degt = degp.T

    hs = _mm1(x, W1, degt)
    s1 = _mp1_kernel(hs.reshape(NC * N, HALF), src2, dst3)
    y2s = _mm2(s1.reshape(NC, N, HALF), degt, b1r, w2p)
    p = _mp2_kernel(y2s, srcw, dstw)
    out = _lsm(p, y2s, degt, b2r)
    return out[:, :7]


# speedup vs baseline: 19.7303x; 1.0558x over previous
import functools

import jax
import jax.numpy as jnp
from jax import lax
from jax.experimental import pallas as pl
from jax.experimental.pallas import tpu as pltpu
from jax.experimental.pallas import tpu_sc as plsc

N = 10000
E = 160000
D = 256
HALF = 128
CP = 16
NC = 2
NS = 16
NT = NC * NS
LANES = 16

RPT = N // NS
CHUNK = 100
CHUNK5 = 250
E_T1 = E // NT
C1 = E_T1 // CHUNK5
E_T3 = E // NS
C3 = E_T3 // CHUNK
RB = 1000
NR = N // RB

_mesh = plsc.VectorSubcoreMesh(
    core_axis_name="c", subcore_axis_name="s", num_cores=NC, num_subcores=NS
)


def _deg_body(dst_hbm, out_hbm, dstv, degv):
    c = lax.axis_index("c")
    s = lax.axis_index("s")
    wid = s * NC + c

    zer = jnp.zeros((LANES,), jnp.float32)

    def zero(i, carry):
        degv[pl.ds(i * LANES, LANES)] = zer
        return carry

    lax.fori_loop(0, N // LANES, zero, 0)

    pltpu.sync_copy(dst_hbm.at[pl.ds(wid * E_T1, E_T1)], dstv)

    ones = jnp.ones((LANES,), jnp.float32)

    def acc(i, carry):
        idx = dstv[pl.ds(i * LANES, LANES)]
        plsc.addupdate_scatter(degv, [idx], ones)
        return carry

    lax.fori_loop(0, E_T1 // LANES, acc, 0)
    pltpu.sync_copy(degv, out_hbm.at[wid])


_deg_kernel = functools.partial(
    pl.kernel,
    out_type=jax.ShapeDtypeStruct((NT, N), jnp.float32),
    mesh=_mesh,
    scratch_types=[
        pltpu.VMEM((E_T1,), jnp.int32),
        pltpu.VMEM((N,), jnp.float32),
    ],
    compiler_params=pltpu.CompilerParams(needs_layout_passes=False),
)(_deg_body)


def _mm1_body(x_ref, w1_ref, degt_ref, hs_ref):
    deg = jnp.sum(degt_ref[...], axis=1, keepdims=True) + 1.0
    dinv = lax.rsqrt(deg)
    prod = jnp.dot(x_ref[...], w1_ref[...], preferred_element_type=jnp.float32)
    hs_ref[...] = (prod * dinv)[None]


def _mm1(x, w1, degt):
    return pl.pallas_call(
        _mm1_body,
        grid=(NR, NC),
        in_specs=[
            pl.BlockSpec((RB, D), lambda r, h: (r, 0)),
            pl.BlockSpec((D, HALF), lambda r, h: (0, h)),
            pl.BlockSpec((RB, NT), lambda r, h: (r, 0)),
        ],
        out_specs=pl.BlockSpec((1, RB, HALF), lambda r, h: (h, r, 0)),
        out_shape=jax.ShapeDtypeStruct((NC, N, HALF), jnp.float32),
    )(x, w1, degt)


def _gs_pipeline(rows_hbm, srcv, dstv, b0, b1, acc, gsem0, gsem1, ssem0,
                 ssem1, nchunks):
    pltpu.async_copy(rows_hbm.at[srcv.at[0]], b0, gsem0)
    pltpu.async_copy(rows_hbm.at[srcv.at[1]], b1, gsem1)

    def step(jj, carry):
        j = 2 * jj
        pltpu.make_async_copy(rows_hbm.at[srcv.at[j]], b0, gsem0).wait()
        pltpu.async_copy(b0, acc.at[dstv.at[j]], ssem0, add=True)
        pltpu.make_async_copy(rows_hbm.at[srcv.at[j + 1]], b1, gsem1).wait()
        pltpu.async_copy(b1, acc.at[dstv.at[j + 1]], ssem1, add=True)

        @pl.when(jj < nchunks // 2 - 1)
        def _():
            pltpu.make_async_copy(b0, acc.at[dstv.at[j]], ssem0).wait()
            pltpu.async_copy(rows_hbm.at[srcv.at[j + 2]], b0, gsem0)
            pltpu.make_async_copy(b1, acc.at[dstv.at[j + 1]], ssem1).wait()
            pltpu.async_copy(rows_hbm.at[srcv.at[j + 3]], b1, gsem1)

        return carry

    lax.fori_loop(0, nchunks // 2, step, 0)
    pltpu.make_async_copy(b0, acc.at[dstv.at[nchunks - 2]], ssem0).wait()
    pltpu.make_async_copy(b1, acc.at[dstv.at[nchunks - 1]], ssem1).wait()


def _mp1_body(hs_hbm, src_hbm, dst_hbm, out_hbm, srcv, dstv, b0, b1, acc,
              gsem0, gsem1, ssem0, ssem1):
    c = lax.axis_index("c")
    s = lax.axis_index("s")

    pltpu.sync_copy(src_hbm.at[c, s], srcv)
    pltpu.sync_copy(dst_hbm.at[s], dstv)
    pltpu.sync_copy(
        hs_hbm.at[pl.ds(c * N + s * RPT, RPT)], acc.at[pl.ds(s * RPT, RPT)]
    )
    plsc.subcore_barrier()
    _gs_pipeline(hs_hbm, srcv, dstv, b0, b1, acc, gsem0, gsem1,
                 ssem0, ssem1, C3)
    plsc.subcore_barrier()
    pltpu.sync_copy(
        acc.at[pl.ds(s * RPT, RPT)], out_hbm.at[pl.ds(c * N + s * RPT, RPT)]
    )


_mp1_kernel = functools.partial(
    pl.kernel,
    out_type=jax.ShapeDtypeStruct((NC * N, HALF), jnp.float32),
    mesh=_mesh,
    scratch_types=[
        pltpu.VMEM((C3, CHUNK), jnp.int32),
        pltpu.VMEM((C3, CHUNK), jnp.int32),
        pltpu.VMEM((CHUNK, HALF), jnp.float32),
        pltpu.VMEM((CHUNK, HALF), jnp.float32),
        pltpu.VMEM_SHARED((N, HALF), jnp.float32),
        pltpu.SemaphoreType.DMA,
        pltpu.SemaphoreType.DMA,
        pltpu.SemaphoreType.DMA,
        pltpu.SemaphoreType.DMA,
    ],
    compiler_params=pltpu.CompilerParams(use_tc_tiling_on_sc=False),
)(_mp1_body)


def _mm2_body(s1_ref, degt_ref, b1_ref, w2_ref, y_ref):
    deg = jnp.sum(degt_ref[...], axis=1, keepdims=True) + 1.0
    dinv = lax.rsqrt(deg)
    s1 = s1_ref[...]
    b1 = b1_ref[...]
    w2 = w2_ref[...]
    ha = jnp.maximum(s1[0] * dinv + b1[0], 0.0)
    hb = jnp.maximum(s1[1] * dinv + b1[1], 0.0)
    y = jnp.dot(ha, w2[0], preferred_element_type=jnp.float32)
    y = y + jnp.dot(hb, w2[1], preferred_element_type=jnp.float32)
    y_ref[...] = y * dinv


def _mm2(s1, degt, b1r, w2r):
    return pl.pallas_call(
        _mm2_body,
        grid=(NR,),
        in_specs=[
            pl.BlockSpec((NC, RB, HALF), lambda r: (0, r, 0)),
            pl.BlockSpec((RB, NT), lambda r: (r, 0)),
            pl.BlockSpec((NC, 1, HALF), lambda r: (0, 0, 0)),
            pl.BlockSpec((NC, HALF, CP), lambda r: (0, 0, 0)),
        ],
        out_specs=pl.BlockSpec((RB, CP), lambda r: (r, 0)),
        out_shape=jax.ShapeDtypeStruct((N, CP), jnp.float32),
    )(s1, degt, b1r, w2r)


def _mp2_body(y_hbm, src_hbm, dst_hbm, out_hbm, srcv, dstv, b0, b1, acc,
              gsem0, gsem1, ssem0, ssem1):
    c = lax.axis_index("c")
    s = lax.axis_index("s")
    wid = s * NC + c

    pltpu.sync_copy(src_hbm.at[wid], srcv)
    pltpu.sync_copy(dst_hbm.at[wid], dstv)

    zer = jnp.zeros((LANES,), jnp.float32)

    def zero(i, carry):
        b0[i] = zer
        return carry

    lax.fori_loop(0, CHUNK5, zero, 0)
    full, rem = divmod(RPT, CHUNK5)
    for k in range(full):
        pltpu.sync_copy(b0, acc.at[pl.ds(s * RPT + k * CHUNK5, CHUNK5)])
    if rem:
        pltpu.sync_copy(
            b0.at[pl.ds(0, rem)],
            acc.at[pl.ds(s * RPT + full * CHUNK5, rem)],
        )
    plsc.subcore_barrier()
    _gs_pipeline(y_hbm, srcv, dstv, b0, b1, acc, gsem0, gsem1,
                 ssem0, ssem1, C1)
    plsc.subcore_barrier()
    pltpu.sync_copy(
        acc.at[pl.ds(s * RPT, RPT)], out_hbm.at[c, pl.ds(s * RPT, RPT)]
    )


_mp2_kernel = functools.partial(
    pl.kernel,
    out_type=jax.ShapeDtypeStruct((NC, N, CP), jnp.float32),
    mesh=_mesh,
    scratch_types=[
        pltpu.VMEM((C1, CHUNK5), jnp.int32),
        pltpu.VMEM((C1, CHUNK5), jnp.int32),
        pltpu.VMEM((CHUNK5, CP), jnp.float32),
        pltpu.VMEM((CHUNK5, CP), jnp.float32),
        pltpu.VMEM_SHARED((N, CP), jnp.float32),
        pltpu.SemaphoreType.DMA,
        pltpu.SemaphoreType.DMA,
        pltpu.SemaphoreType.DMA,
        pltpu.SemaphoreType.DMA,
    ],
    compiler_params=pltpu.CompilerParams(use_tc_tiling_on_sc=False),
)(_mp2_body)


def _lsm_body(p_ref, y_ref, degt_ref, b2_ref, out_ref):
    deg = jnp.sum(degt_ref[...], axis=1, keepdims=True) + 1.0
    dinv = lax.rsqrt(deg)
    p = p_ref[...]
    tot = p[0] + p[1] + y_ref[...]
    logits = tot * dinv + b2_ref[...]
    col = lax.broadcasted_iota(jnp.int32, logits.shape, 1)
    valid = col < 7
    masked = jnp.where(valid, logits, -1e30)
    m = jnp.max(masked, axis=1, keepdims=True)
    z = logits - m
    e = jnp.where(valid, jnp.exp(z), 0.0)
    ssum = jnp.sum(e, axis=1, keepdims=True)
    out_ref[...] = z - jnp.log(ssum)


def _lsm(p, y, degt, b2r):
    return pl.pallas_call(
        _lsm_body,
        grid=(NR,),
        in_specs=[
            pl.BlockSpec((NC, RB, CP), lambda r: (0, r, 0)),
            pl.BlockSpec((RB, CP), lambda r: (r, 0)),
            pl.BlockSpec((RB, NT), lambda r: (r, 0)),
            pl.BlockSpec((1, CP), lambda r: (0, 0)),
        ],
        out_specs=pl.BlockSpec((RB, CP), lambda r: (r, 0)),
        out_shape=jax.ShapeDtypeStruct((N, CP), jnp.float32),
    )(p, y, degt, b2r)


def kernel(x, edge_index, W1, b1, W2, b2):
    ei = edge_index.astype(jnp.int32)
    src = ei[0]
    dst = ei[1]

    src2 = jnp.stack([src, src + N]).reshape(NC, NS, C3, CHUNK)
    dst3 = dst.reshape(NS, C3, CHUNK)
    srcw = src.reshape(NT, C1, CHUNK5)
    dstw = dst.reshape(NT, C1, CHUNK5)

    w2p = jnp.pad(W2, ((0, 0), (0, CP - W2.shape[1]))).reshape(NC, HALF, CP)
    b1r = b1.reshape(NC, 1, HALF)
    b2r = jnp.pad(b2, (0, CP - b2.shape[0])).reshape(1, CP)

    degp = _deg_kernel(dst)
    degt = degp.T

    hs = _mm1(x, W1, degt)
    s1 = _mp1_kernel(hs.reshape(NC * N, HALF), src2, dst3)
    y2s = _mm2(s1.reshape(NC, N, HALF), degt, b1r, w2p)
    p = _mp2_kernel(y2s, srcw, dstw)
    out = _lsm(p, y2s, degt, b2r)
    return out[:, :7]


# speedup vs baseline: 21.8717x; 1.1085x over previous
import functools

import jax
import jax.numpy as jnp
from jax import lax
from jax.experimental import pallas as pl
from jax.experimental.pallas import tpu as pltpu
from jax.experimental.pallas import tpu_sc as plsc

N = 10000
E = 160000
D = 256
HALF = 128
CP = 16
NC = 2
NS = 16
NT = NC * NS
LANES = 16

RPT = N // NS
CHUNK = 50
NB3 = 4
CHUNK5 = 500
E_T1 = E // NT
C1 = E_T1 // CHUNK5
E_T3 = E // NS
C3 = E_T3 // CHUNK
RB = 1000
NR = N // RB

_mesh = plsc.VectorSubcoreMesh(
    core_axis_name="c", subcore_axis_name="s", num_cores=NC, num_subcores=NS
)


def _deg_body(dst_hbm, out_hbm, dstv, degv):
    c = lax.axis_index("c")
    s = lax.axis_index("s")
    wid = s * NC + c

    zer = jnp.zeros((LANES,), jnp.float32)

    def zero(i, carry):
        degv[pl.ds(i * LANES, LANES)] = zer
        return carry

    lax.fori_loop(0, N // LANES, zero, 0)

    pltpu.sync_copy(dst_hbm.at[pl.ds(wid * E_T1, E_T1)], dstv)

    ones = jnp.ones((LANES,), jnp.float32)

    def acc(i, carry):
        idx = dstv[pl.ds(i * LANES, LANES)]
        plsc.addupdate_scatter(degv, [idx], ones)
        return carry

    lax.fori_loop(0, E_T1 // LANES, acc, 0)
    pltpu.sync_copy(degv, out_hbm.at[wid])


_deg_kernel = functools.partial(
    pl.kernel,
    out_type=jax.ShapeDtypeStruct((NT, N), jnp.float32),
    mesh=_mesh,
    scratch_types=[
        pltpu.VMEM((E_T1,), jnp.int32),
        pltpu.VMEM((N,), jnp.float32),
    ],
    compiler_params=pltpu.CompilerParams(needs_layout_passes=False),
)(_deg_body)


def _mm1_body(x_ref, w1_ref, degt_ref, hs_ref):
    deg = jnp.sum(degt_ref[...], axis=1, keepdims=True) + 1.0
    dinv = lax.rsqrt(deg)
    prod = jnp.dot(x_ref[...], w1_ref[...], preferred_element_type=jnp.float32)
    hs_ref[...] = (prod * dinv)[None]


def _mm1(x, w1, degt):
    return pl.pallas_call(
        _mm1_body,
        grid=(NR, NC),
        in_specs=[
            pl.BlockSpec((RB, D), lambda r, h: (r, 0)),
            pl.BlockSpec((D, HALF), lambda r, h: (0, h)),
            pl.BlockSpec((RB, NT), lambda r, h: (r, 0)),
        ],
        out_specs=pl.BlockSpec((1, RB, HALF), lambda r, h: (h, r, 0)),
        out_shape=jax.ShapeDtypeStruct((NC, N, HALF), jnp.float32),
    )(x, w1, degt)


def _gs_pipeline(rows_hbm, srcv, dstv, bufs, acc, gsems, ssems, nchunks):
    nb = len(bufs)
    for k in range(nb):
        pltpu.async_copy(rows_hbm.at[srcv.at[k]], bufs[k], gsems[k])

    def step(jj, carry):
        j = nb * jj
        for k in range(nb):
            pltpu.make_async_copy(
                rows_hbm.at[srcv.at[j + k]], bufs[k], gsems[k]).wait()
            pltpu.async_copy(bufs[k], acc.at[dstv.at[j + k]], ssems[k],
                             add=True)

        @pl.when(jj < nchunks // nb - 1)
        def _():
            for k in range(nb):
                pltpu.make_async_copy(
                    bufs[k], acc.at[dstv.at[j + k]], ssems[k]).wait()
                pltpu.async_copy(
                    rows_hbm.at[srcv.at[j + nb + k]], bufs[k], gsems[k])

        return carry

    lax.fori_loop(0, nchunks // nb, step, 0)
    for k in range(nb):
        pltpu.make_async_copy(
            bufs[k], acc.at[dstv.at[nchunks - nb + k]], ssems[k]).wait()


def _mp1_body(hs_hbm, src_hbm, dst_hbm, out_hbm, srcv, dstv, b0, b1, b2,
              b3, acc, gsem0, gsem1, gsem2, gsem3, ssem0, ssem1, ssem2,
              ssem3):
    c = lax.axis_index("c")
    s = lax.axis_index("s")

    pltpu.sync_copy(src_hbm.at[c, s], srcv)
    pltpu.sync_copy(dst_hbm.at[s], dstv)
    pltpu.sync_copy(
        hs_hbm.at[pl.ds(c * N + s * RPT, RPT)], acc.at[pl.ds(s * RPT, RPT)]
    )
    plsc.subcore_barrier()
    _gs_pipeline(hs_hbm, srcv, dstv, [b0, b1, b2, b3], acc,
                 [gsem0, gsem1, gsem2, gsem3],
                 [ssem0, ssem1, ssem2, ssem3], C3)
    plsc.subcore_barrier()
    pltpu.sync_copy(
        acc.at[pl.ds(s * RPT, RPT)], out_hbm.at[pl.ds(c * N + s * RPT, RPT)]
    )


_mp1_kernel = functools.partial(
    pl.kernel,
    out_type=jax.ShapeDtypeStruct((NC * N, HALF), jnp.float32),
    mesh=_mesh,
    scratch_types=[
        pltpu.VMEM((C3, CHUNK), jnp.int32),
        pltpu.VMEM((C3, CHUNK), jnp.int32),
        pltpu.VMEM((CHUNK, HALF), jnp.float32),
        pltpu.VMEM((CHUNK, HALF), jnp.float32),
        pltpu.VMEM((CHUNK, HALF), jnp.float32),
        pltpu.VMEM((CHUNK, HALF), jnp.float32),
        pltpu.VMEM_SHARED((N, HALF), jnp.float32),
        pltpu.SemaphoreType.DMA,
        pltpu.SemaphoreType.DMA,
        pltpu.SemaphoreType.DMA,
        pltpu.SemaphoreType.DMA,
        pltpu.SemaphoreType.DMA,
        pltpu.SemaphoreType.DMA,
        pltpu.SemaphoreType.DMA,
        pltpu.SemaphoreType.DMA,
    ],
    compiler_params=pltpu.CompilerParams(use_tc_tiling_on_sc=False),
)(_mp1_body)


def _mm2_body(s1_ref, degt_ref, b1_ref, w2_ref, y_ref):
    deg = jnp.sum(degt_ref[...], axis=1, keepdims=True) + 1.0
    dinv = lax.rsqrt(deg)
    s1 = s1_ref[...]
    b1 = b1_ref[...]
    w2 = w2_ref[...]
    ha = jnp.maximum(s1[0] * dinv + b1[0], 0.0)
    hb = jnp.maximum(s1[1] * dinv + b1[1], 0.0)
    y = jnp.dot(ha, w2[0], preferred_element_type=jnp.float32)
    y = y + jnp.dot(hb, w2[1], preferred_element_type=jnp.float32)
    y_ref[...] = y * dinv


def _mm2(s1, degt, b1r, w2r):
    return pl.pallas_call(
        _mm2_body,
        grid=(NR,),
        in_specs=[
            pl.BlockSpec((NC, RB, HALF), lambda r: (0, r, 0)),
            pl.BlockSpec((RB, NT), lambda r: (r, 0)),
            pl.BlockSpec((NC, 1, HALF), lambda r: (0, 0, 0)),
            pl.BlockSpec((NC, HALF, CP), lambda r: (0, 0, 0)),
        ],
        out_specs=pl.BlockSpec((RB, CP), lambda r: (r, 0)),
        out_shape=jax.ShapeDtypeStruct((N, CP), jnp.float32),
    )(s1, degt, b1r, w2r)


def _mp2_body(y_hbm, src_hbm, dst_hbm, out_hbm, srcv, dstv, b0, b1, acc,
              gsem0, gsem1, ssem0, ssem1):
    c = lax.axis_index("c")
    s = lax.axis_index("s")
    wid = s * NC + c

    pltpu.sync_copy(src_hbm.at[wid], srcv)
    pltpu.sync_copy(dst_hbm.at[wid], dstv)

    zer = jnp.zeros((LANES,), jnp.float32)

    def zero(i, carry):
        b0[i] = zer
        return carry

    lax.fori_loop(0, CHUNK5, zero, 0)
    full, rem = divmod(RPT, CHUNK5)
    for k in range(full):
        pltpu.sync_copy(b0, acc.at[pl.ds(s * RPT + k * CHUNK5, CHUNK5)])
    if rem:
        pltpu.sync_copy(
            b0.at[pl.ds(0, rem)],
            acc.at[pl.ds(s * RPT + full * CHUNK5, rem)],
        )
    plsc.subcore_barrier()
    _gs_pipeline(y_hbm, srcv, dstv, [b0, b1], acc, [gsem0, gsem1],
                 [ssem0, ssem1], C1)
    plsc.subcore_barrier()
    pltpu.sync_copy(
        acc.at[pl.ds(s * RPT, RPT)], out_hbm.at[c, pl.ds(s * RPT, RPT)]
    )


_mp2_kernel = functools.partial(
    pl.kernel,
    out_type=jax.ShapeDtypeStruct((NC, N, CP), jnp.float32),
    mesh=_mesh,
    scratch_types=[
        pltpu.VMEM((C1, CHUNK5), jnp.int32),
        pltpu.VMEM((C1, CHUNK5), jnp.int32),
        pltpu.VMEM((CHUNK5, CP), jnp.float32),
        pltpu.VMEM((CHUNK5, CP), jnp.float32),
        pltpu.VMEM_SHARED((N, CP), jnp.float32),
        pltpu.SemaphoreType.DMA,
        pltpu.SemaphoreType.DMA,
        pltpu.SemaphoreType.DMA,
        pltpu.SemaphoreType.DMA,
    ],
    compiler_params=pltpu.CompilerParams(use_tc_tiling_on_sc=False),
)(_mp2_body)


def _lsm_body(p_ref, y_ref, degt_ref, b2_ref, out_ref):
    deg = jnp.sum(degt_ref[...], axis=1, keepdims=True) + 1.0
    dinv = lax.rsqrt(deg)
    p = p_ref[...]
    tot = p[0] + p[1] + y_ref[...]
    logits = tot * dinv + b2_ref[...]
    col = lax.broadcasted_iota(jnp.int32, logits.shape, 1)
    valid = col < 7
    masked = jnp.where(valid, logits, -1e30)
    m = jnp.max(masked, axis=1, keepdims=True)
    z = logits - m
    e = jnp.where(valid, jnp.exp(z), 0.0)
    ssum = jnp.sum(e, axis=1, keepdims=True)
    out_ref[...] = z - jnp.log(ssum)


def _lsm(p, y, degt, b2r):
    return pl.pallas_call(
        _lsm_body,
        grid=(NR,),
        in_specs=[
            pl.BlockSpec((NC, RB, CP), lambda r: (0, r, 0)),
            pl.BlockSpec((RB, CP), lambda r: (r, 0)),
            pl.BlockSpec((RB, NT), lambda r: (r, 0)),
            pl.BlockSpec((1, CP), lambda r: (0, 0)),
        ],
        out_specs=pl.BlockSpec((RB, CP), lambda r: (r, 0)),
        out_shape=jax.ShapeDtypeStruct((N, CP), jnp.float32),
    )(p, y, degt, b2r)


def kernel(x, edge_index, W1, b1, W2, b2):
    ei = edge_index.astype(jnp.int32)
    src = ei[0]
    dst = ei[1]

    src2 = jnp.stack([src, src + N]).reshape(NC, NS, C3, CHUNK)
    dst3 = dst.reshape(NS, C3, CHUNK)
    srcw = src.reshape(NT, C1, CHUNK5)
    dstw = dst.reshape(NT, C1, CHUNK5)

    w2p = jnp.pad(W2, ((0, 0), (0, CP - W2.shape[1]))).reshape(NC, HALF, CP)
    b1r = b1.reshape(NC, 1, HALF)
    b2r = jnp.pad(b2, (0, CP - b2.shape[0])).reshape(1, CP)

    degp = _deg_kernel(dst)
    degt = degp.T

    hs = _mm1(x, W1, degt)
    s1 = _mp1_kernel(hs.reshape(NC * N, HALF), src2, dst3)
    y2s = _mm2(s1.reshape(NC, N, HALF), degt, b1r, w2p)
    p = _mp2_kernel(y2s, srcw, dstw)
    out = _lsm(p, y2s, degt, b2r)
    return out[:, :7]


# speedup vs baseline: 22.7083x; 1.0382x over previous
import functools

import jax
import jax.numpy as jnp
from jax import lax
from jax.experimental import pallas as pl
from jax.experimental.pallas import tpu as pltpu
from jax.experimental.pallas import tpu_sc as plsc

N = 10000
E = 160000
D = 256
HALF = 128
CP = 16
NC = 2
NS = 16
NT = NC * NS
LANES = 16

RPT = N // NS
CHUNK = 50
NB3 = 4
CHUNK5 = 625
E_T1 = E // NT
C1 = E_T1 // CHUNK5
E_T3 = E // NS
C3 = E_T3 // CHUNK
RB = 1000
NR = N // RB

_mesh = plsc.VectorSubcoreMesh(
    core_axis_name="c", subcore_axis_name="s", num_cores=NC, num_subcores=NS
)


def _deg_body(dst_hbm, out_hbm, dstv, degv):
    c = lax.axis_index("c")
    s = lax.axis_index("s")
    wid = s * NC + c

    zer = jnp.zeros((LANES,), jnp.float32)

    def zero(i, carry):
        degv[pl.ds(i * LANES, LANES)] = zer
        return carry

    lax.fori_loop(0, N // LANES, zero, 0)

    pltpu.sync_copy(dst_hbm.at[pl.ds(wid * E_T1, E_T1)], dstv)

    ones = jnp.ones((LANES,), jnp.float32)

    def acc(i, carry):
        idx = dstv[pl.ds(i * LANES, LANES)]
        plsc.addupdate_scatter(degv, [idx], ones)
        return carry

    lax.fori_loop(0, E_T1 // LANES, acc, 0)
    pltpu.sync_copy(degv, out_hbm.at[wid])


_deg_kernel = functools.partial(
    pl.kernel,
    out_type=jax.ShapeDtypeStruct((NT, N), jnp.float32),
    mesh=_mesh,
    scratch_types=[
        pltpu.VMEM((E_T1,), jnp.int32),
        pltpu.VMEM((N,), jnp.float32),
    ],
    compiler_params=pltpu.CompilerParams(needs_layout_passes=False),
)(_deg_body)


def _mm1_body(x_ref, w1_ref, degt_ref, hs_ref):
    deg = jnp.sum(degt_ref[...], axis=1, keepdims=True) + 1.0
    dinv = lax.rsqrt(deg)
    x = x_ref[...]
    w1 = w1_ref[...]
    hs_ref[0] = jnp.dot(x, w1[:, :HALF],
                        preferred_element_type=jnp.float32) * dinv
    hs_ref[1] = jnp.dot(x, w1[:, HALF:],
                        preferred_element_type=jnp.float32) * dinv


def _mm1(x, w1, degt):
    return pl.pallas_call(
        _mm1_body,
        grid=(NR,),
        in_specs=[
            pl.BlockSpec((RB, D), lambda r: (r, 0)),
            pl.BlockSpec((D, D), lambda r: (0, 0)),
            pl.BlockSpec((RB, NT), lambda r: (r, 0)),
        ],
        out_specs=pl.BlockSpec((NC, RB, HALF), lambda r: (0, r, 0)),
        out_shape=jax.ShapeDtypeStruct((NC, N, HALF), jnp.float32),
    )(x, w1, degt)


def _gs_pipeline(rows_hbm, srcv, dstv, bufs, acc, gsems, ssems, nchunks):
    nb = len(bufs)
    for k in range(nb):
        pltpu.async_copy(rows_hbm.at[srcv.at[k]], bufs[k], gsems[k])

    def step(jj, carry):
        j = nb * jj
        for k in range(nb):
            pltpu.make_async_copy(
                rows_hbm.at[srcv.at[j + k]], bufs[k], gsems[k]).wait()
            pltpu.async_copy(bufs[k], acc.at[dstv.at[j + k]], ssems[k],
                             add=True)

        @pl.when(jj < nchunks // nb - 1)
        def _():
            for k in range(nb):
                pltpu.make_async_copy(
                    bufs[k], acc.at[dstv.at[j + k]], ssems[k]).wait()
                pltpu.async_copy(
                    rows_hbm.at[srcv.at[j + nb + k]], bufs[k], gsems[k])

        return carry

    lax.fori_loop(0, nchunks // nb, step, 0)
    for k in range(nb):
        pltpu.make_async_copy(
            bufs[k], acc.at[dstv.at[nchunks - nb + k]], ssems[k]).wait()


def _mp1_body(hs_hbm, src_hbm, dst_hbm, out_hbm, srcv, dstv, b0, b1, b2,
              b3, acc, gsem0, gsem1, gsem2, gsem3, ssem0, ssem1, ssem2,
              ssem3):
    c = lax.axis_index("c")
    s = lax.axis_index("s")

    pltpu.sync_copy(src_hbm.at[c, s], srcv)
    pltpu.sync_copy(dst_hbm.at[s], dstv)
    pltpu.sync_copy(
        hs_hbm.at[pl.ds(c * N + s * RPT, RPT)], acc.at[pl.ds(s * RPT, RPT)]
    )
    plsc.subcore_barrier()
    _gs_pipeline(hs_hbm, srcv, dstv, [b0, b1, b2, b3], acc,
                 [gsem0, gsem1, gsem2, gsem3],
                 [ssem0, ssem1, ssem2, ssem3], C3)
    plsc.subcore_barrier()
    pltpu.sync_copy(
        acc.at[pl.ds(s * RPT, RPT)], out_hbm.at[pl.ds(c * N + s * RPT, RPT)]
    )


_mp1_kernel = functools.partial(
    pl.kernel,
    out_type=jax.ShapeDtypeStruct((NC * N, HALF), jnp.float32),
    mesh=_mesh,
    scratch_types=[
        pltpu.VMEM((C3, CHUNK), jnp.int32),
        pltpu.VMEM((C3, CHUNK), jnp.int32),
        pltpu.VMEM((CHUNK, HALF), jnp.float32),
        pltpu.VMEM((CHUNK, HALF), jnp.float32),
        pltpu.VMEM((CHUNK, HALF), jnp.float32),
        pltpu.VMEM((CHUNK, HALF), jnp.float32),
        pltpu.VMEM_SHARED((N, HALF), jnp.float32),
        pltpu.SemaphoreType.DMA,
        pltpu.SemaphoreType.DMA,
        pltpu.SemaphoreType.DMA,
        pltpu.SemaphoreType.DMA,
        pltpu.SemaphoreType.DMA,
        pltpu.SemaphoreType.DMA,
        pltpu.SemaphoreType.DMA,
        pltpu.SemaphoreType.DMA,
    ],
    compiler_params=pltpu.CompilerParams(use_tc_tiling_on_sc=False),
)(_mp1_body)


def _mm2_body(s1_ref, degt_ref, b1_ref, w2_ref, y_ref):
    deg = jnp.sum(degt_ref[...], axis=1, keepdims=True) + 1.0
    dinv = lax.rsqrt(deg)
    s1 = s1_ref[...]
    b1 = b1_ref[...]
    w2 = w2_ref[...]
    ha = jnp.maximum(s1[0] * dinv + b1[0], 0.0)
    hb = jnp.maximum(s1[1] * dinv + b1[1], 0.0)
    y = jnp.dot(ha, w2[0], preferred_element_type=jnp.float32)
    y = y + jnp.dot(hb, w2[1], preferred_element_type=jnp.float32)
    y_ref[...] = y * dinv


def _mm2(s1, degt, b1r, w2r):
    return pl.pallas_call(
        _mm2_body,
        grid=(NR,),
        in_specs=[
            pl.BlockSpec((NC, RB, HALF), lambda r: (0, r, 0)),
            pl.BlockSpec((RB, NT), lambda r: (r, 0)),
            pl.BlockSpec((NC, 1, HALF), lambda r: (0, 0, 0)),
            pl.BlockSpec((NC, HALF, CP), lambda r: (0, 0, 0)),
        ],
        out_specs=pl.BlockSpec((RB, CP), lambda r: (r, 0)),
        out_shape=jax.ShapeDtypeStruct((N, CP), jnp.float32),
    )(s1, degt, b1r, w2r)


def _mp2_body(y_hbm, src_hbm, dst_hbm, out_hbm, srcv, dstv, b0, b1, acc,
              gsem0, gsem1, ssem0, ssem1):
    c = lax.axis_index("c")
    s = lax.axis_index("s")
    wid = s * NC + c

    pltpu.sync_copy(src_hbm.at[wid], srcv)
    pltpu.sync_copy(dst_hbm.at[wid], dstv)

    zer = jnp.zeros((LANES,), jnp.float32)

    def zero(i, carry):
        b0[i] = zer
        return carry

    lax.fori_loop(0, CHUNK5, zero, 0)
    full, rem = divmod(RPT, CHUNK5)
    for k in range(full):
        pltpu.sync_copy(b0, acc.at[pl.ds(s * RPT + k * CHUNK5, CHUNK5)])
    if rem:
        pltpu.sync_copy(
            b0.at[pl.ds(0, rem)],
            acc.at[pl.ds(s * RPT + full * CHUNK5, rem)],
        )
    plsc.subcore_barrier()
    _gs_pipeline(y_hbm, srcv, dstv, [b0, b1], acc, [gsem0, gsem1],
                 [ssem0, ssem1], C1)
    plsc.subcore_barrier()
    pltpu.sync_copy(
        acc.at[pl.ds(s * RPT, RPT)], out_hbm.at[c, pl.ds(s * RPT, RPT)]
    )


_mp2_kernel = functools.partial(
    pl.kernel,
    out_type=jax.ShapeDtypeStruct((NC, N, CP), jnp.float32),
    mesh=_mesh,
    scratch_types=[
        pltpu.VMEM((C1, CHUNK5), jnp.int32),
        pltpu.VMEM((C1, CHUNK5), jnp.int32),
        pltpu.VMEM((CHUNK5, CP), jnp.float32),
        pltpu.VMEM((CHUNK5, CP), jnp.float32),
        pltpu.VMEM_SHARED((N, CP), jnp.float32),
        pltpu.SemaphoreType.DMA,
        pltpu.SemaphoreType.DMA,
        pltpu.SemaphoreType.DMA,
        pltpu.SemaphoreType.DMA,
    ],
    compiler_params=pltpu.CompilerParams(use_tc_tiling_on_sc=False),
)(_mp2_body)


def _lsm_body(p_ref, y_ref, degt_ref, b2_ref, out_ref):
    deg = jnp.sum(degt_ref[...], axis=1, keepdims=True) + 1.0
    dinv = lax.rsqrt(deg)
    p = p_ref[...]
    tot = p[0] + p[1] + y_ref[...]
    logits = tot * dinv + b2_ref[...]
    col = lax.broadcasted_iota(jnp.int32, logits.shape, 1)
    valid = col < 7
    masked = jnp.where(valid, logits, -1e30)
    m = jnp.max(masked, axis=1, keepdims=True)
    z = logits - m
    e = jnp.where(valid, jnp.exp(z), 0.0)
    ssum = jnp.sum(e, axis=1, keepdims=True)
    out_ref[...] = z - jnp.log(ssum)


def _lsm(p, y, degt, b2r):
    return pl.pallas_call(
        _lsm_body,
        grid=(NR,),
        in_specs=[
            pl.BlockSpec((NC, RB, CP), lambda r: (0, r, 0)),
            pl.BlockSpec((RB, CP), lambda r: (r, 0)),
            pl.BlockSpec((RB, NT), lambda r: (r, 0)),
            pl.BlockSpec((1, CP), lambda r: (0, 0)),
        ],
        out_specs=pl.BlockSpec((RB, CP), lambda r: (r, 0)),
        out_shape=jax.ShapeDtypeStruct((N, CP), jnp.float32),
    )(p, y, degt, b2r)


def kernel(x, edge_index, W1, b1, W2, b2):
    ei = edge_index.astype(jnp.int32)
    src = ei[0]
    dst = ei[1]

    src2 = jnp.stack([src, src + N]).reshape(NC, NS, C3, CHUNK)
    dst3 = dst.reshape(NS, C3, CHUNK)
    srcw = src.reshape(NT, C1, CHUNK5)
    dstw = dst.reshape(NT, C1, CHUNK5)

    w2p = jnp.pad(W2, ((0, 0), (0, CP - W2.shape[1]))).reshape(NC, HALF, CP)
    b1r = b1.reshape(NC, 1, HALF)
    b2r = jnp.pad(b2, (0, CP - b2.shape[0])).reshape(1, CP)

    degp = _deg_kernel(dst)
    degt = degp.T

    hs = _mm1(x, W1, degt)
    s1 = _mp1_kernel(hs.reshape(NC * N, HALF), src2, dst3)
    y2s = _mm2(s1.reshape(NC, N, HALF), degt, b1r, w2p)
    p = _mp2_kernel(y2s, srcw, dstw)
    out = _lsm(p, y2s, degt, b2r)
    return out[:, :7]


# speedup vs baseline: 23.1982x; 1.0216x over previous
import functools

import jax
import jax.numpy as jnp
from jax import lax
from jax.experimental import pallas as pl
from jax.experimental.pallas import tpu as pltpu
from jax.experimental.pallas import tpu_sc as plsc

N = 10000
E = 160000
D = 256
HALF = 128
CP = 16
NC = 2
NS = 16
NT = NC * NS
LANES = 16

RPT = N // NS
CHUNK = 50
NB3 = 5
NPH = 4
CHUNK5 = 625
E_T1 = E // NT
C1 = E_T1 // CHUNK5
E_T3 = E // NS
C3 = E_T3 // CHUNK
PH = C3 // NPH
RB = 1000
NR = N // RB

_mesh = plsc.VectorSubcoreMesh(
    core_axis_name="c", subcore_axis_name="s", num_cores=NC, num_subcores=NS
)


def _deg_body(dst_hbm, out_hbm, dstv, degv):
    c = lax.axis_index("c")
    s = lax.axis_index("s")
    wid = s * NC + c

    zer = jnp.zeros((LANES,), jnp.float32)

    def zero(i, carry):
        degv[pl.ds(i * LANES, LANES)] = zer
        return carry

    lax.fori_loop(0, N // LANES, zero, 0)

    pltpu.sync_copy(dst_hbm.at[pl.ds(wid * E_T1, E_T1)], dstv)

    ones = jnp.ones((LANES,), jnp.float32)

    def acc(i, carry):
        idx = dstv[pl.ds(i * LANES, LANES)]
        plsc.addupdate_scatter(degv, [idx], ones)
        return carry

    lax.fori_loop(0, E_T1 // LANES, acc, 0)
    pltpu.sync_copy(degv, out_hbm.at[wid])


_deg_kernel = functools.partial(
    pl.kernel,
    out_type=jax.ShapeDtypeStruct((NT, N), jnp.float32),
    mesh=_mesh,
    scratch_types=[
        pltpu.VMEM((E_T1,), jnp.int32),
        pltpu.VMEM((N,), jnp.float32),
    ],
    compiler_params=pltpu.CompilerParams(needs_layout_passes=False),
)(_deg_body)


def _mm1_body(x_ref, w1_ref, degt_ref, hs_ref):
    deg = jnp.sum(degt_ref[...], axis=1, keepdims=True) + 1.0
    dinv = lax.rsqrt(deg)
    x = x_ref[...]
    w1 = w1_ref[...]
    hs_ref[0] = jnp.dot(x, w1[:, :HALF],
                        preferred_element_type=jnp.float32) * dinv
    hs_ref[1] = jnp.dot(x, w1[:, HALF:],
                        preferred_element_type=jnp.float32) * dinv


def _mm1(x, w1, degt):
    return pl.pallas_call(
        _mm1_body,
        grid=(NR,),
        in_specs=[
            pl.BlockSpec((RB, D), lambda r: (r, 0)),
            pl.BlockSpec((D, D), lambda r: (0, 0)),
            pl.BlockSpec((RB, NT), lambda r: (r, 0)),
        ],
        out_specs=pl.BlockSpec((NC, RB, HALF), lambda r: (0, r, 0)),
        out_shape=jax.ShapeDtypeStruct((NC, N, HALF), jnp.float32),
    )(x, w1, degt)


def _gs_pipeline(rows_hbm, srcv, dstv, bufs, acc, gsems, ssems, nchunks):
    nb = len(bufs)
    for k in range(nb):
        pltpu.async_copy(rows_hbm.at[srcv.at[k]], bufs[k], gsems[k])

    def step(jj, carry):
        j = nb * jj
        for k in range(nb):
            pltpu.make_async_copy(
                rows_hbm.at[srcv.at[j + k]], bufs[k], gsems[k]).wait()
            pltpu.async_copy(bufs[k], acc.at[dstv.at[j + k]], ssems[k],
                             add=True)

        @pl.when(jj < nchunks // nb - 1)
        def _():
            for k in range(nb):
                pltpu.make_async_copy(
                    bufs[k], acc.at[dstv.at[j + k]], ssems[k]).wait()
                pltpu.async_copy(
                    rows_hbm.at[srcv.at[j + nb + k]], bufs[k], gsems[k])

        return carry

    lax.fori_loop(0, nchunks // nb, step, 0)
    for k in range(nb):
        pltpu.make_async_copy(
            bufs[k], acc.at[dstv.at[nchunks - nb + k]], ssems[k]).wait()


def _mp1_body(hs_hbm, src_hbm, dst_hbm, out_hbm, sv0, sv1, dv0, dv1, b0,
              b1, b2, b3, b4, acc, isem0, isem1, gsem0, gsem1, gsem2,
              gsem3, gsem4, ssem0, ssem1, ssem2, ssem3, ssem4):
    c = lax.axis_index("c")
    s = lax.axis_index("s")
    svs, dvs, isems = [sv0, sv1], [dv0, dv1], [isem0, isem1]
    bufs = [b0, b1, b2, b3, b4]
    gsems = [gsem0, gsem1, gsem2, gsem3, gsem4]
    ssems = [ssem0, ssem1, ssem2, ssem3, ssem4]

    pltpu.async_copy(src_hbm.at[c, s, 0], sv0, isem0)
    pltpu.async_copy(dst_hbm.at[s, 0], dv0, isem0)
    pltpu.sync_copy(
        hs_hbm.at[pl.ds(c * N + s * RPT, RPT)], acc.at[pl.ds(s * RPT, RPT)]
    )
    pltpu.make_async_copy(src_hbm.at[c, s, 0], sv0, isem0).wait()
    pltpu.make_async_copy(dst_hbm.at[s, 0], dv0, isem0).wait()
    plsc.subcore_barrier()

    for ph in range(NPH):
        if ph + 1 < NPH:
            nxt = (ph + 1) % 2
            pltpu.async_copy(src_hbm.at[c, s, ph + 1], svs[nxt], isems[nxt])
            pltpu.async_copy(dst_hbm.at[s, ph + 1], dvs[nxt], isems[nxt])
        _gs_pipeline(hs_hbm, svs[ph % 2], dvs[ph % 2], bufs, acc, gsems,
                     ssems, PH)
        if ph + 1 < NPH:
            nxt = (ph + 1) % 2
            pltpu.make_async_copy(
                src_hbm.at[c, s, ph + 1], svs[nxt], isems[nxt]).wait()
            pltpu.make_async_copy(
                dst_hbm.at[s, ph + 1], dvs[nxt], isems[nxt]).wait()

    plsc.subcore_barrier()
    pltpu.sync_copy(
        acc.at[pl.ds(s * RPT, RPT)], out_hbm.at[pl.ds(c * N + s * RPT, RPT)]
    )


_mp1_kernel = functools.partial(
    pl.kernel,
    out_type=jax.ShapeDtypeStruct((NC * N, HALF), jnp.float32),
    mesh=_mesh,
    scratch_types=(
        [pltpu.VMEM((PH, CHUNK), jnp.int32)] * 4
        + [pltpu.VMEM((CHUNK, HALF), jnp.float32)] * 5
        + [pltpu.VMEM_SHARED((N, HALF), jnp.float32)]
        + [pltpu.SemaphoreType.DMA] * 12
    ),
    compiler_params=pltpu.CompilerParams(use_tc_tiling_on_sc=False),
)(_mp1_body)


def _mm2_body(s1_ref, degt_ref, b1_ref, w2_ref, y_ref):
    deg = jnp.sum(degt_ref[...], axis=1, keepdims=True) + 1.0
    dinv = lax.rsqrt(deg)
    s1 = s1_ref[...]
    b1 = b1_ref[...]
    w2 = w2_ref[...]
    ha = jnp.maximum(s1[0] * dinv + b1[0], 0.0)
    hb = jnp.maximum(s1[1] * dinv + b1[1], 0.0)
    y = jnp.dot(ha, w2[0], preferred_element_type=jnp.float32)
    y = y + jnp.dot(hb, w2[1], preferred_element_type=jnp.float32)
    y_ref[...] = y * dinv


def _mm2(s1, degt, b1r, w2r):
    return pl.pallas_call(
        _mm2_body,
        grid=(NR,),
        in_specs=[
            pl.BlockSpec((NC, RB, HALF), lambda r: (0, r, 0)),
            pl.BlockSpec((RB, NT), lambda r: (r, 0)),
            pl.BlockSpec((NC, 1, HALF), lambda r: (0, 0, 0)),
            pl.BlockSpec((NC, HALF, CP), lambda r: (0, 0, 0)),
        ],
        out_specs=pl.BlockSpec((RB, CP), lambda r: (r, 0)),
        out_shape=jax.ShapeDtypeStruct((N, CP), jnp.float32),
    )(s1, degt, b1r, w2r)


def _mp2_body(y_hbm, src_hbm, dst_hbm, out_hbm, srcv, dstv, b0, b1, acc,
              gsem0, gsem1, ssem0, ssem1):
    c = lax.axis_index("c")
    s = lax.axis_index("s")
    wid = s * NC + c

    pltpu.sync_copy(src_hbm.at[wid], srcv)
    pltpu.sync_copy(dst_hbm.at[wid], dstv)

    zer = jnp.zeros((LANES,), jnp.float32)

    def zero(i, carry):
        b0[i] = zer
        return carry

    lax.fori_loop(0, CHUNK5, zero, 0)
    full, rem = divmod(RPT, CHUNK5)
    for k in range(full):
        pltpu.sync_copy(b0, acc.at[pl.ds(s * RPT + k * CHUNK5, CHUNK5)])
    if rem:
        pltpu.sync_copy(
            b0.at[pl.ds(0, rem)],
            acc.at[pl.ds(s * RPT + full * CHUNK5, rem)],
        )
    plsc.subcore_barrier()
    _gs_pipeline(y_hbm, srcv, dstv, [b0, b1], acc, [gsem0, gsem1],
                 [ssem0, ssem1], C1)
    plsc.subcore_barrier()
    pltpu.sync_copy(
        acc.at[pl.ds(s * RPT, RPT)], out_hbm.at[c, pl.ds(s * RPT, RPT)]
    )


_mp2_kernel = functools.partial(
    pl.kernel,
    out_type=jax.ShapeDtypeStruct((NC, N, CP), jnp.float32),
    mesh=_mesh,
    scratch_types=[
        pltpu.VMEM((C1, CHUNK5), jnp.int32),
        pltpu.VMEM((C1, CHUNK5), jnp.int32),
        pltpu.VMEM((CHUNK5, CP), jnp.float32),
        pltpu.VMEM((CHUNK5, CP), jnp.float32),
        pltpu.VMEM_SHARED((N, CP), jnp.float32),
        pltpu.SemaphoreType.DMA,
        pltpu.SemaphoreType.DMA,
        pltpu.SemaphoreType.DMA,
        pltpu.SemaphoreType.DMA,
    ],
    compiler_params=pltpu.CompilerParams(use_tc_tiling_on_sc=False),
)(_mp2_body)


def _lsm_body(p_ref, y_ref, degt_ref, b2_ref, out_ref):
    deg = jnp.sum(degt_ref[...], axis=1, keepdims=True) + 1.0
    dinv = lax.rsqrt(deg)
    p = p_ref[...]
    tot = p[0] + p[1] + y_ref[...]
    logits = tot * dinv + b2_ref[...]
    col = lax.broadcasted_iota(jnp.int32, logits.shape, 1)
    valid = col < 7
    masked = jnp.where(valid, logits, -1e30)
    m = jnp.max(masked, axis=1, keepdims=True)
    z = logits - m
    e = jnp.where(valid, jnp.exp(z), 0.0)
    ssum = jnp.sum(e, axis=1, keepdims=True)
    out_ref[...] = z - jnp.log(ssum)


def _lsm(p, y, degt, b2r):
    return pl.pallas_call(
        _lsm_body,
        grid=(NR,),
        in_specs=[
            pl.BlockSpec((NC, RB, CP), lambda r: (0, r, 0)),
            pl.BlockSpec((RB, CP), lambda r: (r, 0)),
            pl.BlockSpec((RB, NT), lambda r: (r, 0)),
            pl.BlockSpec((1, CP), lambda r: (0, 0)),
        ],
        out_specs=pl.BlockSpec((RB, CP), lambda r: (r, 0)),
        out_shape=jax.ShapeDtypeStruct((N, CP), jnp.float32),
    )(p, y, degt, b2r)


def kernel(x, edge_index, W1, b1, W2, b2):
    ei = edge_index.astype(jnp.int32)
    src = ei[0]
    dst = ei[1]

    src2 = jnp.stack([src, src + N]).reshape(NC, NS, NPH, PH, CHUNK)
    dst3 = dst.reshape(NS, NPH, PH, CHUNK)
    srcw = src.reshape(NT, C1, CHUNK5)
    dstw = dst.reshape(NT, C1, CHUNK5)

    w2p = jnp.pad(W2, ((0, 0), (0, CP - W2.shape[1]))).reshape(NC, HALF, CP)
    b1r = b1.reshape(NC, 1, HALF)
    b2r = jnp.pad(b2, (0, CP - b2.shape[0])).reshape(1, CP)

    degp = _deg_kernel(dst)
    degt = degp.T

    hs = _mm1(x, W1, degt)
    s1 = _mp1_kernel(hs.reshape(NC * N, HALF), src2, dst3)
    y2s = _mm2(s1.reshape(NC, N, HALF), degt, b1r, w2p)
    p = _mp2_kernel(y2s, srcw, dstw)
    out = _lsm(p, y2s, degt, b2r)
    return out[:, :7]


# speedup vs baseline: 23.3788x; 1.0078x over previous
import functools

import jax
import jax.numpy as jnp
from jax import lax
from jax.experimental import pallas as pl
from jax.experimental.pallas import tpu as pltpu
from jax.experimental.pallas import tpu_sc as plsc

N = 10000
E = 160000
D = 256
HALF = 128
CP = 16
NC = 2
NS = 16
NT = NC * NS
LANES = 16

RPT = N // NS
CHUNK = 50
NB3 = 5
NPH = 4
CHUNK5 = 625
E_T1 = E // NT
C1 = E_T1 // CHUNK5
E_T3 = E // NS
C3 = E_T3 // CHUNK
PH = C3 // NPH
RB = 1000
NR = N // RB

_mesh = plsc.VectorSubcoreMesh(
    core_axis_name="c", subcore_axis_name="s", num_cores=NC, num_subcores=NS
)


def _deg_body(dst_hbm, out_hbm, dstv, degv, isem):
    c = lax.axis_index("c")
    s = lax.axis_index("s")
    wid = s * NC + c

    pltpu.async_copy(dst_hbm.at[pl.ds(wid * E_T1, E_T1)], dstv, isem)
    zer = jnp.zeros((LANES,), jnp.float32)

    def zero(i, carry):
        degv[pl.ds(i * LANES, LANES)] = zer
        return carry

    lax.fori_loop(0, N // LANES, zero, 0)

    pltpu.make_async_copy(
        dst_hbm.at[pl.ds(wid * E_T1, E_T1)], dstv, isem).wait()

    ones = jnp.ones((LANES,), jnp.float32)

    def acc(i, carry):
        idx = dstv[pl.ds(i * LANES, LANES)]
        plsc.addupdate_scatter(degv, [idx], ones)
        return carry

    lax.fori_loop(0, E_T1 // LANES, acc, 0)
    pltpu.sync_copy(degv, out_hbm.at[wid])


_deg_kernel = functools.partial(
    pl.kernel,
    out_type=jax.ShapeDtypeStruct((NT, N), jnp.float32),
    mesh=_mesh,
    scratch_types=[
        pltpu.VMEM((E_T1,), jnp.int32),
        pltpu.VMEM((N,), jnp.float32),
        pltpu.SemaphoreType.DMA,
    ],
    compiler_params=pltpu.CompilerParams(needs_layout_passes=False),
)(_deg_body)


def _mm1_body(x_ref, w1_ref, degt_ref, hs_ref):
    deg = jnp.sum(degt_ref[...], axis=1, keepdims=True) + 1.0
    dinv = lax.rsqrt(deg)
    x = x_ref[...]
    w1 = w1_ref[...]
    hs_ref[0] = jnp.dot(x, w1[:, :HALF],
                        preferred_element_type=jnp.float32) * dinv
    hs_ref[1] = jnp.dot(x, w1[:, HALF:],
                        preferred_element_type=jnp.float32) * dinv


def _mm1(x, w1, degt):
    return pl.pallas_call(
        _mm1_body,
        grid=(NR,),
        in_specs=[
            pl.BlockSpec((RB, D), lambda r: (r, 0)),
            pl.BlockSpec((D, D), lambda r: (0, 0)),
            pl.BlockSpec((RB, NT), lambda r: (r, 0)),
        ],
        out_specs=pl.BlockSpec((NC, RB, HALF), lambda r: (0, r, 0)),
        out_shape=jax.ShapeDtypeStruct((NC, N, HALF), jnp.float32),
    )(x, w1, degt)


def _gs_pipeline(rows_hbm, srcv, dstv, bufs, acc, gsems, ssems, nchunks):
    nb = len(bufs)
    for k in range(nb):
        pltpu.async_copy(rows_hbm.at[srcv.at[k]], bufs[k], gsems[k])

    def step(jj, carry):
        j = nb * jj
        for k in range(nb):
            pltpu.make_async_copy(
                rows_hbm.at[srcv.at[j + k]], bufs[k], gsems[k]).wait()
            pltpu.async_copy(bufs[k], acc.at[dstv.at[j + k]], ssems[k],
                             add=True)

        @pl.when(jj < nchunks // nb - 1)
        def _():
            for k in range(nb):
                pltpu.make_async_copy(
                    bufs[k], acc.at[dstv.at[j + k]], ssems[k]).wait()
                pltpu.async_copy(
                    rows_hbm.at[srcv.at[j + nb + k]], bufs[k], gsems[k])

        return carry

    lax.fori_loop(0, nchunks // nb, step, 0)
    for k in range(nb):
        pltpu.make_async_copy(
            bufs[k], acc.at[dstv.at[nchunks - nb + k]], ssems[k]).wait()


def _mp1_body(hs_hbm, src_hbm, dst_hbm, out_hbm, sv0, sv1, dv0, dv1, b0,
              b1, b2, b3, b4, acc, isem0, isem1, gsem0, gsem1, gsem2,
              gsem3, gsem4, ssem0, ssem1, ssem2, ssem3, ssem4):
    c = lax.axis_index("c")
    s = lax.axis_index("s")
    svs, dvs, isems = [sv0, sv1], [dv0, dv1], [isem0, isem1]
    bufs = [b0, b1, b2, b3, b4]
    gsems = [gsem0, gsem1, gsem2, gsem3, gsem4]
    ssems = [ssem0, ssem1, ssem2, ssem3, ssem4]

    pltpu.async_copy(src_hbm.at[c, s, 0], sv0, isem0)
    pltpu.async_copy(dst_hbm.at[s, 0], dv0, isem0)
    pltpu.sync_copy(
        hs_hbm.at[pl.ds(c * N + s * RPT, RPT)], acc.at[pl.ds(s * RPT, RPT)]
    )
    pltpu.make_async_copy(src_hbm.at[c, s, 0], sv0, isem0).wait()
    pltpu.make_async_copy(dst_hbm.at[s, 0], dv0, isem0).wait()
    plsc.subcore_barrier()

    for ph in range(NPH):
        if ph + 1 < NPH:
            nxt = (ph + 1) % 2
            pltpu.async_copy(src_hbm.at[c, s, ph + 1], svs[nxt], isems[nxt])
            pltpu.async_copy(dst_hbm.at[s, ph + 1], dvs[nxt], isems[nxt])
        _gs_pipeline(hs_hbm, svs[ph % 2], dvs[ph % 2], bufs, acc, gsems,
                     ssems, PH)
        if ph + 1 < NPH:
            nxt = (ph + 1) % 2
            pltpu.make_async_copy(
                src_hbm.at[c, s, ph + 1], svs[nxt], isems[nxt]).wait()
            pltpu.make_async_copy(
                dst_hbm.at[s, ph + 1], dvs[nxt], isems[nxt]).wait()

    plsc.subcore_barrier()
    pltpu.sync_copy(
        acc.at[pl.ds(s * RPT, RPT)], out_hbm.at[pl.ds(c * N + s * RPT, RPT)]
    )


_mp1_kernel = functools.partial(
    pl.kernel,
    out_type=jax.ShapeDtypeStruct((NC * N, HALF), jnp.float32),
    mesh=_mesh,
    scratch_types=(
        [pltpu.VMEM((PH, CHUNK), jnp.int32)] * 4
        + [pltpu.VMEM((CHUNK, HALF), jnp.float32)] * 5
        + [pltpu.VMEM_SHARED((N, HALF), jnp.float32)]
        + [pltpu.SemaphoreType.DMA] * 12
    ),
    compiler_params=pltpu.CompilerParams(use_tc_tiling_on_sc=False),
)(_mp1_body)


def _mm2_body(s1_ref, degt_ref, b1_ref, w2_ref, y_ref):
    deg = jnp.sum(degt_ref[...], axis=1, keepdims=True) + 1.0
    dinv = lax.rsqrt(deg)
    s1 = s1_ref[...]
    b1 = b1_ref[...]
    w2 = w2_ref[...]
    ha = jnp.maximum(s1[0] * dinv + b1[0], 0.0)
    hb = jnp.maximum(s1[1] * dinv + b1[1], 0.0)
    y = jnp.dot(ha, w2[0], preferred_element_type=jnp.float32)
    y = y + jnp.dot(hb, w2[1], preferred_element_type=jnp.float32)
    y_ref[...] = y * dinv


def _mm2(s1, degt, b1r, w2r):
    return pl.pallas_call(
        _mm2_body,
        grid=(NR,),
        in_specs=[
            pl.BlockSpec((NC, RB, HALF), lambda r: (0, r, 0)),
            pl.BlockSpec((RB, NT), lambda r: (r, 0)),
            pl.BlockSpec((NC, 1, HALF), lambda r: (0, 0, 0)),
            pl.BlockSpec((NC, HALF, CP), lambda r: (0, 0, 0)),
        ],
        out_specs=pl.BlockSpec((RB, CP), lambda r: (r, 0)),
        out_shape=jax.ShapeDtypeStruct((N, CP), jnp.float32),
    )(s1, degt, b1r, w2r)


def _mp2_body(y_hbm, src_hbm, dst_hbm, out_hbm, srcv, dstv, b0, b1, acc,
              gsem0, gsem1, ssem0, ssem1):
    c = lax.axis_index("c")
    s = lax.axis_index("s")
    wid = s * NC + c

    pltpu.async_copy(src_hbm.at[wid], srcv, gsem0)
    pltpu.async_copy(dst_hbm.at[wid], dstv, gsem1)

    zer = jnp.zeros((LANES,), jnp.float32)

    def zero(i, carry):
        b0[i] = zer
        return carry

    lax.fori_loop(0, CHUNK5, zero, 0)
    full, rem = divmod(RPT, CHUNK5)
    for k in range(full):
        pltpu.sync_copy(b0, acc.at[pl.ds(s * RPT + k * CHUNK5, CHUNK5)])
    if rem:
        pltpu.sync_copy(
            b0.at[pl.ds(0, rem)],
            acc.at[pl.ds(s * RPT + full * CHUNK5, rem)],
        )
    pltpu.make_async_copy(src_hbm.at[wid], srcv, gsem0).wait()
    pltpu.make_async_copy(dst_hbm.at[wid], dstv, gsem1).wait()
    plsc.subcore_barrier()
    _gs_pipeline(y_hbm, srcv, dstv, [b0, b1], acc, [gsem0, gsem1],
                 [ssem0, ssem1], C1)
    plsc.subcore_barrier()
    pltpu.sync_copy(
        acc.at[pl.ds(s * RPT, RPT)], out_hbm.at[c, pl.ds(s * RPT, RPT)]
    )


_mp2_kernel = functools.partial(
    pl.kernel,
    out_type=jax.ShapeDtypeStruct((NC, N, CP), jnp.float32),
    mesh=_mesh,
    scratch_types=[
        pltpu.VMEM((C1, CHUNK5), jnp.int32),
        pltpu.VMEM((C1, CHUNK5), jnp.int32),
        pltpu.VMEM((CHUNK5, CP), jnp.float32),
        pltpu.VMEM((CHUNK5, CP), jnp.float32),
        pltpu.VMEM_SHARED((N, CP), jnp.float32),
        pltpu.SemaphoreType.DMA,
        pltpu.SemaphoreType.DMA,
        pltpu.SemaphoreType.DMA,
        pltpu.SemaphoreType.DMA,
    ],
    compiler_params=pltpu.CompilerParams(use_tc_tiling_on_sc=False),
)(_mp2_body)


def _lsm_body(p_ref, y_ref, degt_ref, b2_ref, out_ref):
    deg = jnp.sum(degt_ref[...], axis=1, keepdims=True) + 1.0
    dinv = lax.rsqrt(deg)
    p = p_ref[...]
    tot = p[0] + p[1] + y_ref[...]
    logits = tot * dinv + b2_ref[...]
    col = lax.broadcasted_iota(jnp.int32, logits.shape, 1)
    valid = col < 7
    masked = jnp.where(valid, logits, -1e30)
    m = jnp.max(masked, axis=1, keepdims=True)
    z = logits - m
    e = jnp.where(valid, jnp.exp(z), 0.0)
    ssum = jnp.sum(e, axis=1, keepdims=True)
    out_ref[...] = (z - jnp.log(ssum))[:, :7]


def _lsm(p, y, degt, b2r):
    return pl.pallas_call(
        _lsm_body,
        grid=(NR,),
        in_specs=[
            pl.BlockSpec((NC, RB, CP), lambda r: (0, r, 0)),
            pl.BlockSpec((RB, CP), lambda r: (r, 0)),
            pl.BlockSpec((RB, NT), lambda r: (r, 0)),
            pl.BlockSpec((1, CP), lambda r: (0, 0)),
        ],
        out_specs=pl.BlockSpec((RB, 7), lambda r: (r, 0)),
        out_shape=jax.ShapeDtypeStruct((N, 7), jnp.float32),
    )(p, y, degt, b2r)


def kernel(x, edge_index, W1, b1, W2, b2):
    ei = edge_index.astype(jnp.int32)
    src = ei[0]
    dst = ei[1]

    src2 = jnp.stack([src, src + N]).reshape(NC, NS, NPH, PH, CHUNK)
    dst3 = dst.reshape(NS, NPH, PH, CHUNK)
    srcw = src.reshape(NT, C1, CHUNK5)
    dstw = dst.reshape(NT, C1, CHUNK5)

    w2p = jnp.pad(W2, ((0, 0), (0, CP - W2.shape[1]))).reshape(NC, HALF, CP)
    b1r = b1.reshape(NC, 1, HALF)
    b2r = jnp.pad(b2, (0, CP - b2.shape[0])).reshape(1, CP)

    degp = _deg_kernel(dst)
    degt = degp.T

    hs = _mm1(x, W1, degt)
    s1 = _mp1_kernel(hs.reshape(NC * N, HALF), src2, dst3)
    y2s = _mm2(s1.reshape(NC, N, HALF), degt, b1r, w2p)
    p = _mp2_kernel(y2s, srcw, dstw)
    return _lsm(p, y2s, degt, b2r)
